# baseline (speedup 1.0000x reference)
import functools

import jax
import jax.numpy as jnp
from jax import lax
from jax.experimental import pallas as pl
from jax.experimental.pallas import tpu as pltpu
from jax.experimental.pallas import tpu_sc as plsc


@functools.lru_cache(maxsize=None)
def _make_relayout(V, D, dtype_name):
    dtype = jnp.dtype(dtype_name)
    info = plsc.get_sparse_core_info()
    NW = info.num_cores * info.num_subcores
    NC = info.num_cores
    L = 128
    VT = V // L
    TAIL = V - VT * L
    VPAD = VT * L + (L if TAIL else 0)
    RPB = L * D // L
    NBUF = 3
    OUTER = (VT + NW * NBUF - 1) // (NW * NBUF)

    mesh = plsc.VectorSubcoreMesh(core_axis_name="c", subcore_axis_name="s")

    @functools.partial(
        pl.kernel,
        mesh=mesh,
        out_type=jax.ShapeDtypeStruct((VPAD * D // L, L), dtype),
        scratch_types=[
            pltpu.VMEM((NBUF, D, L), dtype),
            pltpu.VMEM((NBUF, RPB, L), dtype),
            pltpu.SemaphoreType.DMA((NBUF,)),
            pltpu.SemaphoreType.DMA((NBUF,)),
        ],
        compiler_params=pltpu.CompilerParams(
            use_tc_tiling_on_sc=True, needs_layout_passes=False
        ),
    )
    def relayout_kernel(tabt_hbm, out_hbm, inbuf, tbuf, isem, osem):
        wid = lax.axis_index("s") * NC + lax.axis_index("c")
        nfull = (VT - wid + NW - 1) // NW

        iota = lax.iota(jnp.int32, 16)

        def in_copy(vt, b):
            return pltpu.make_async_copy(
                tabt_hbm.at[:, pl.ds(vt * L, L)], inbuf.at[b], isem.at[b]
            )

        def out_copy(vt, b):
            return pltpu.make_async_copy(
                tbuf.at[b], out_hbm.at[pl.ds(vt * RPB, RPB)], osem.at[b]
            )

        def transpose(b):
            @plsc.parallel_loop(0, L, 1, unroll=8)
            def t_body(v):
                col = jnp.full((16,), v, jnp.int32)
                for c in range(D // 16):
                    val = plsc.load_gather(inbuf.at[b], [16 * c + iota, col])
                    off = v * D + 16 * c
                    tbuf[b, off >> 7, pl.ds(off & (L - 1), 16)] = val

        for b in range(NBUF):
            @pl.when(b < nfull)
            def _():
                in_copy(wid + b * NW, b).start()

        def body(i, carry):
            for b in range(NBUF):
                g = i * NBUF + b

                @pl.when(g < nfull)
                def _():
                    vt = wid + g * NW
                    in_copy(vt, b).wait()

                    @pl.when(g >= NBUF)
                    def _():
                        out_copy(wid + (g - NBUF) * NW, b).wait()

                    transpose(b)
                    out_copy(vt, b).start()

                    @pl.when(g + NBUF < nfull)
                    def _():
                        in_copy(wid + (g + NBUF) * NW, b).start()
            return carry

        lax.fori_loop(0, OUTER, body, 0)

        for b in range(NBUF):
            @pl.when(nfull > b)
            def _():
                out_copy(wid, b).wait()

    return relayout_kernel


@functools.lru_cache(maxsize=None)
def _make_gather(BATCH, HIST, VROWS, D, dtype_name):
    dtype = jnp.dtype(dtype_name)
    info = plsc.get_sparse_core_info()
    NW = info.num_cores * info.num_subcores
    NC = info.num_cores
    L = 128
    SUB = 8
    assert D % SUB == 0 and BATCH % (L * NW) == 0
    DT = D // SUB
    KB = BATCH // (L * NW)
    PITCH = L + 1

    mesh = plsc.VectorSubcoreMesh(core_axis_name="c", subcore_axis_name="s")

    @functools.partial(
        pl.kernel,
        mesh=mesh,
        out_type=jax.ShapeDtypeStruct((HIST, DT, BATCH // L, SUB, L), dtype),
        scratch_types=[
            pltpu.VMEM((KB, HIST, L), jnp.int32),
            pltpu.VMEM((KB, L, D), dtype),
            pltpu.VMEM((KB, D, PITCH), dtype),
            pltpu.SemaphoreType.DMA((KB,)),
            pltpu.SemaphoreType.DMA((KB,)),
            pltpu.SemaphoreType.DMA,
        ],
        compiler_params=pltpu.CompilerParams(
            use_tc_tiling_on_sc=False, needs_layout_passes=False
        ),
    )
    def gather_kernel(idx_hbm, table_hbm, out_hbm, islab, gbuf, tbuf,
                      gsem, osem, ssem):
        wid = lax.axis_index("s") * NC + lax.axis_index("c")

        iota = lax.iota(jnp.int32, 16)

        for k in range(KB):
            pltpu.async_copy(
                idx_hbm.at[:, pl.ds(L * (KB * wid + k), L)],
                islab.at[k], ssem,
            ).wait()

        def gather_block(h, k):
            return pltpu.make_async_copy(
                table_hbm.at[islab.at[k, h]],
                gbuf.at[k],
                gsem.at[k],
            )

        def out_copy(h, k, dt):
            bt = KB * wid + k
            return pltpu.make_async_copy(
                tbuf.at[k, pl.ds(SUB * dt, SUB), pl.ds(0, L)],
                out_hbm.at[h, dt, bt],
                osem.at[k],
            )

        for k in range(KB):
            gather_block(0, k).start()

        row_ids = [16 * c + iota for c in range(D // 16)]

        def h_body(h, carry):
            for k in range(KB):
                gather_block(h, k).wait()

                @pl.when(h > 0)
                def _():
                    for dt in range(DT):
                        out_copy(h - 1, k, dt).wait()

                @plsc.parallel_loop(0, L, 1, unroll=8)
                def t_body(bl):
                    col = jnp.full((16,), bl, jnp.int32)
                    for c in range(D // 16):
                        v = gbuf[k, bl, pl.ds(16 * c, 16)]
                        plsc.store_scatter(
                            tbuf.at[k], [row_ids[c], col], v
                        )

                @pl.when(h + 1 < HIST)
                def _():
                    gather_block(h + 1, k).start()

                for dt in range(DT):
                    out_copy(h, k, dt).start()
            return carry

        lax.fori_loop(0, HIST, h_body, 0)

        for k in range(KB):
            for dt in range(DT):
                out_copy(HIST - 1, k, dt).wait()

    return gather_kernel


def kernel(indices, table):
    BATCH, HIST = indices.shape
    V, D = table.shape
    L = 128
    VT = V // L
    TAIL = V - VT * L
    VPAD = VT * L + (L if TAIL else 0)

    idx_t = indices.astype(jnp.int32).T

    relayout = _make_relayout(V, D, str(table.dtype))
    scratch = relayout(table.T)
    if TAIL:
        tail_vals = table[VT * L:].reshape(TAIL * D // L, L)
        scratch = lax.dynamic_update_slice(scratch, tail_vals, (VT * D, 0))
    table2 = scratch.reshape(VPAD, D)

    gather = _make_gather(BATCH, HIST, VPAD, D, str(table.dtype))
    out5 = gather(idx_t, table2)
    return out5.transpose(2, 4, 0, 1, 3).reshape(BATCH, HIST, D)


# speedup vs baseline: 1.3791x; 1.3791x over previous
import functools

import jax
import jax.numpy as jnp
from jax import lax
from jax.experimental import pallas as pl
from jax.experimental.pallas import tpu as pltpu
from jax.experimental.pallas import tpu_sc as plsc


@functools.lru_cache(maxsize=None)
def _make_gather(BATCH, HIST, VROWS, D, STRIDE, dtype_name):
    dtype = jnp.dtype(dtype_name)
    info = plsc.get_sparse_core_info()
    NW = info.num_cores * info.num_subcores
    NC = info.num_cores
    L = 128
    SUB = 8
    assert D % SUB == 0 and BATCH % (L * NW) == 0
    DT = D // SUB
    KB = BATCH // (L * NW)
    PITCH = L + 1

    mesh = plsc.VectorSubcoreMesh(core_axis_name="c", subcore_axis_name="s")

    @functools.partial(
        pl.kernel,
        mesh=mesh,
        out_type=jax.ShapeDtypeStruct((HIST, DT, BATCH // L, SUB, L), dtype),
        scratch_types=[
            pltpu.VMEM((KB, HIST, L), jnp.int32),
            pltpu.VMEM((KB, L, D), dtype),
            pltpu.VMEM((KB, D, PITCH), dtype),
            pltpu.SemaphoreType.DMA((KB,)),
            pltpu.SemaphoreType.DMA((KB,)),
            pltpu.SemaphoreType.DMA,
        ],
        compiler_params=pltpu.CompilerParams(
            use_tc_tiling_on_sc=False, needs_layout_passes=False
        ),
    )
    def gather_kernel(idx_hbm, table_hbm, out_hbm, islab, gbuf, tbuf,
                      gsem, osem, ssem):
        wid = lax.axis_index("s") * NC + lax.axis_index("c")

        iota = lax.iota(jnp.int32, 16)

        for k in range(KB):
            pltpu.async_copy(
                idx_hbm.at[:, pl.ds(L * (KB * wid + k), L)],
                islab.at[k], ssem,
            ).wait()

        if STRIDE > 1:
            for k in range(KB):
                @plsc.parallel_loop(0, HIST * L // 16, 1, unroll=8)
                def _dbl(i):
                    h = i >> 3
                    off = (i & 7) * 16
                    v = islab[k, h, pl.ds(off, 16)]
                    islab[k, h, pl.ds(off, 16)] = v * STRIDE

        def gather_block(h, k):
            return pltpu.make_async_copy(
                table_hbm.at[islab.at[k, h]],
                gbuf.at[k],
                gsem.at[k],
            )

        def out_copy(h, k, dt):
            bt = KB * wid + k
            return pltpu.make_async_copy(
                tbuf.at[k, pl.ds(SUB * dt, SUB), pl.ds(0, L)],
                out_hbm.at[h, dt, bt],
                osem.at[k],
            )

        for k in range(KB):
            gather_block(0, k).start()

        row_ids = [16 * c + iota for c in range(D // 16)]

        def h_body(h, carry):
            for k in range(KB):
                gather_block(h, k).wait()

                @pl.when(h > 0)
                def _():
                    for dt in range(DT):
                        out_copy(h - 1, k, dt).wait()

                @plsc.parallel_loop(0, L, 1, unroll=8)
                def t_body(bl):
                    col = jnp.full((16,), bl, jnp.int32)
                    for c in range(D // 16):
                        v = gbuf[k, bl, pl.ds(16 * c, 16)]
                        plsc.store_scatter(
                            tbuf.at[k], [row_ids[c], col], v
                        )

                @pl.when(h + 1 < HIST)
                def _():
                    gather_block(h + 1, k).start()

                for dt in range(DT):
                    out_copy(h, k, dt).start()
            return carry

        lax.fori_loop(0, HIST, h_body, 0)

        for k in range(KB):
            for dt in range(DT):
                out_copy(HIST - 1, k, dt).wait()

    return gather_kernel


def kernel(indices, table):
    BATCH, HIST = indices.shape
    V, D = table.shape
    L = 128

    idx_t = indices.astype(jnp.int32).T

    stride = L // D if L % D == 0 and L != D else 1
    if stride > 1:
        tpad = jnp.pad(table, ((0, 0), (0, L - D)))
        table2 = tpad.reshape(V * stride, D)
    else:
        table2 = table

    gather = _make_gather(BATCH, HIST, table2.shape[0], D, stride,
                          str(table.dtype))
    out5 = gather(idx_t, table2)
    return out5.transpose(2, 4, 0, 1, 3).reshape(BATCH, HIST, D)
